# baseline (device time: 28725 ns/iter reference)
import jax
import jax.numpy as jnp
from jax import lax
from jax.experimental import pallas as pl
from jax.experimental.pallas import tpu as pltpu

N_DEV = 4
M, K, N = 2048, 2048, 2048
MP = M // N_DEV
KP = K // N_DEV


def kernel(x, w_mat):

    def body(x_hbm, w_hbm, out_ref, xf_ref, xbf_ref, gat_ref, wf_ref,
             wbl_ref, x_sems, w_sems, send_sems, recv_sems):
        my = lax.axis_index("i")

        x_dmas = []
        for idx, d in enumerate((1, 2, 3, 0)):
            k = lax.rem(my + d, N_DEV)
            blk = pl.ds(k * MP, MP)
            dma = pltpu.make_async_copy(
                x_hbm.at[blk, :], xf_ref.at[blk, :], x_sems.at[idx])
            dma.start()
            x_dmas.append(dma)
        w_dmas = []
        for idx, off in enumerate((0, 3, 1, 2)):
            k = lax.rem(my + off, N_DEV)
            dma = pltpu.make_async_copy(
                w_hbm.at[pl.ds(k * KP, KP), :], wf_ref.at[k], w_sems.at[idx])
            dma.start()
            w_dmas.append(dma)

        barrier_sem = pltpu.get_barrier_semaphore()
        for d in range(1, N_DEV):
            peer = lax.rem(my + d, N_DEV)
            pl.semaphore_signal(
                barrier_sem, inc=1,
                device_id=(peer,), device_id_type=pl.DeviceIdType.MESH,
            )
        pl.semaphore_wait(barrier_sem, N_DEV - 1)

        sends = []
        for i, d in enumerate((1, 2, 3)):
            peer = lax.rem(my + d, N_DEV)
            blk = pl.ds(peer * MP, MP)
            x_dmas[i].wait()
            xbf_ref[blk, :] = xf_ref[blk, :].astype(jnp.bfloat16)
            rdma = pltpu.make_async_remote_copy(
                src_ref=xbf_ref.at[blk, :],
                dst_ref=gat_ref.at[my],
                send_sem=send_sems.at[i],
                recv_sem=recv_sems.at[i],
                device_id=(peer,),
                device_id_type=pl.DeviceIdType.MESH,
            )
            rdma.start()
            sends.append(rdma)

        x_dmas[3].wait()
        gat_ref[my] = xf_ref[pl.ds(my * MP, MP), :].astype(jnp.bfloat16)
        w_dmas[0].wait()
        acc = jnp.dot(
            gat_ref[my], wf_ref[my].astype(jnp.bfloat16),
            preferred_element_type=jnp.float32,
        )

        kdiag = lax.rem(my + 2, N_DEV)
        w_dmas[3].wait()
        wbl_ref[:, :] = wf_ref[kdiag].astype(jnp.bfloat16)

        for wi, d in ((1, 1), (2, 3)):
            src = lax.rem(my - d + N_DEV, N_DEV)
            recv = pltpu.make_async_remote_copy(
                src_ref=xbf_ref.at[pl.ds(0, MP), :],
                dst_ref=gat_ref.at[src],
                send_sem=send_sems.at[d - 1],
                recv_sem=recv_sems.at[d - 1],
                device_id=(src,),
                device_id_type=pl.DeviceIdType.MESH,
            )
            recv.wait_recv()
            w_dmas[wi].wait()
            acc = acc + jnp.dot(
                gat_ref[src], wf_ref[src].astype(jnp.bfloat16),
                preferred_element_type=jnp.float32,
            )

        recv = pltpu.make_async_remote_copy(
            src_ref=xbf_ref.at[pl.ds(0, MP), :],
            dst_ref=gat_ref.at[kdiag],
            send_sem=send_sems.at[1],
            recv_sem=recv_sems.at[1],
            device_id=(kdiag,),
            device_id_type=pl.DeviceIdType.MESH,
        )
        recv.wait_recv()
        acc = acc + jnp.dot(
            gat_ref[kdiag], wbl_ref[:, :], preferred_element_type=jnp.float32,
        )

        out_ref[:, :] = acc * jax.nn.sigmoid(acc)

        for rdma in sends:
            rdma.wait_send()

    return pl.pallas_call(
        body,
        out_shape=jax.ShapeDtypeStruct((MP, N), jnp.float32),
        in_specs=[
            pl.BlockSpec(memory_space=pl.ANY),
            pl.BlockSpec(memory_space=pl.ANY),
        ],
        out_specs=pl.BlockSpec(memory_space=pltpu.VMEM),
        scratch_shapes=[
            pltpu.VMEM((M, KP), jnp.float32),
            pltpu.VMEM((M, KP), jnp.bfloat16),
            pltpu.VMEM((N_DEV, MP, KP), jnp.bfloat16),
            pltpu.VMEM((N_DEV, KP, N), jnp.float32),
            pltpu.VMEM((KP, N), jnp.bfloat16),
            pltpu.SemaphoreType.DMA((N_DEV,)),
            pltpu.SemaphoreType.DMA((N_DEV,)),
            pltpu.SemaphoreType.DMA((N_DEV - 1,)),
            pltpu.SemaphoreType.DMA((N_DEV - 1,)),
        ],
        compiler_params=pltpu.CompilerParams(
            collective_id=0, vmem_limit_bytes=64 * 1024 * 1024,
        ),
    )(x, w_mat)
